# baseline (speedup 1.0000x reference)
import functools

import jax
import jax.numpy as jnp
from jax import lax
from jax.experimental import pallas as pl
from jax.experimental.pallas import tpu as pltpu
from jax.experimental.pallas import tpu_sc as plsc

BATCH = 1024
MAXLEN = 200
EMBED_DIM = 64
LANES = 16
NUM_WORKERS = 32
SEQ_PER_WORKER = BATCH // NUM_WORKERS
CHUNKS = ((0, 104), (104, 96))
NBUF = 2


def _embed_body(x_hbm, tok_hbm, pos_hbm, out_hbm,
                pos_v, idx0, idx1, rows0, rows1, g0, g1, s0, s1):
    nc = 2
    wid = lax.axis_index("s") * nc + lax.axis_index("c")
    base = wid * SEQ_PER_WORKER

    pltpu.sync_copy(pos_hbm, pos_v)

    bufs = ((idx0, rows0, g0, s0), (idx1, rows1, g1, s1))

    def start_seq(i, buf):
        idx, rows, gsem, _ = buf
        seq = base + i
        pltpu.sync_copy(x_hbm.at[seq], idx)
        cps = []
        for off, ln in CHUNKS:
            cps.append(pltpu.async_copy(
                tok_hbm.at[idx.at[pl.ds(off, ln)]],
                rows.at[pl.ds(off, ln)], gsem))
        return cps

    inflight = [None] * NBUF
    store_cp = [None] * NBUF
    inflight[0] = start_seq(0, bufs[0])
    for i in range(SEQ_PER_WORKER):
        cur = i % NBUF
        nxt = (i + 1) % NBUF
        if i + 1 < SEQ_PER_WORKER:
            if store_cp[nxt] is not None:
                store_cp[nxt].wait()
            inflight[nxt] = start_seq(i + 1, bufs[nxt])
        for cp in inflight[cur]:
            cp.wait()
        rows = bufs[cur][1]

        def add_pos(r, carry):
            for j in range(EMBED_DIM // LANES):
                sl = pl.ds(j * LANES, LANES)
                plsc.addupdate(rows.at[r, sl], pos_v[r, sl])
            return carry

        lax.fori_loop(0, MAXLEN, add_pos, 0)
        store_cp[cur] = pltpu.async_copy(rows, out_hbm.at[base + i],
                                         bufs[cur][3])
    for cp in store_cp:
        if cp is not None:
            cp.wait()


@jax.jit
def _embed(x, token_table, pos_table):
    mesh = plsc.VectorSubcoreMesh(core_axis_name="c", subcore_axis_name="s")
    run = functools.partial(
        pl.kernel, mesh=mesh,
        out_type=jax.ShapeDtypeStruct((BATCH, MAXLEN, EMBED_DIM), jnp.float32),
        scratch_types=[
            pltpu.VMEM((MAXLEN, EMBED_DIM), jnp.float32),
            pltpu.VMEM((MAXLEN,), jnp.int32),
            pltpu.VMEM((MAXLEN,), jnp.int32),
            pltpu.VMEM((MAXLEN, EMBED_DIM), jnp.float32),
            pltpu.VMEM((MAXLEN, EMBED_DIM), jnp.float32),
            pltpu.SemaphoreType.DMA,
            pltpu.SemaphoreType.DMA,
            pltpu.SemaphoreType.DMA,
            pltpu.SemaphoreType.DMA,
        ],
        compiler_params=pltpu.CompilerParams(use_tc_tiling_on_sc=False),
    )(_embed_body)
    return run(x, token_table, pos_table)


def kernel(x, token_table, pos_table):
    return _embed(x.astype(jnp.int32), token_table, pos_table)
